# baseline (device time: 135911 ns/iter reference)
import jax
import jax.numpy as jnp
from jax import lax
from jax.experimental import pallas as pl
from jax.experimental.pallas import tpu as pltpu

N_DEV = 4


def kernel(x, w_mat, scale_x, scale_w):
    m_tot, k_loc = x.shape
    k_tot, n_tot = w_mat.shape
    m_loc = m_tot // N_DEV
    assert m_loc == k_loc

    BN = 1024
    n_blocks = n_tot // BN

    def body(x_ref, w_ref, sx_ref, sw_ref, out_ref,
             x8_ref, gat_ref, send_sems, recv_sems):
        n = pl.program_id(0)
        j = pl.program_id(1)
        my_i = lax.axis_index("i")

        @pl.when(jnp.logical_and(n == 0, j == 0))
        def _init():
            x8_ref[:, :] = x_ref[:, :].astype(jnp.float8_e4m3fn)
            gat_ref[my_i] = x8_ref[pl.ds(my_i * m_loc, m_loc), :]
            rdmas = []
            for d in range(1, N_DEV):
                peer = lax.rem(my_i + d, N_DEV)
                rdma = pltpu.make_async_remote_copy(
                    src_ref=x8_ref.at[pl.ds(peer * m_loc, m_loc), :],
                    dst_ref=gat_ref.at[my_i],
                    send_sem=send_sems.at[peer],
                    recv_sem=recv_sems.at[my_i],
                    device_id=(peer,),
                    device_id_type=pl.DeviceIdType.MESH,
                )
                rdma.start()
                rdmas.append(rdma)
            for rdma in rdmas:
                rdma.wait_send()

        @pl.when(jnp.logical_and(n == 0, j != my_i))
        def _wait():
            recv = pltpu.make_async_remote_copy(
                src_ref=gat_ref.at[j],
                dst_ref=gat_ref.at[j],
                send_sem=send_sems.at[j],
                recv_sem=recv_sems.at[j],
                device_id=(j,),
                device_id_type=pl.DeviceIdType.MESH,
            )
            recv.wait_recv()

        w8 = w_ref[:, :].astype(jnp.float8_e5m2)
        acc = lax.dot_general(
            gat_ref[j], w8,
            dimension_numbers=(((1,), (0,)), ((), ())),
            preferred_element_type=jnp.float32,
        )

        @pl.when(j == 0)
        def _store():
            out_ref[:, :] = acc

        @pl.when(j != 0)
        def _accum():
            out_ref[:, :] = out_ref[:, :] + acc

        @pl.when(j == N_DEV - 1)
        def _epilogue():
            s = sx_ref[0] * sw_ref[0]
            out_ref[:, :] = jnp.maximum(out_ref[:, :] * s, 0.0)

    return pl.pallas_call(
        body,
        grid=(n_blocks, N_DEV),
        in_specs=[
            pl.BlockSpec(memory_space=pltpu.VMEM),
            pl.BlockSpec((k_loc, BN), lambda n, j: (j, n)),
            pl.BlockSpec(memory_space=pltpu.SMEM),
            pl.BlockSpec(memory_space=pltpu.SMEM),
        ],
        out_specs=pl.BlockSpec((m_loc, BN), lambda n, j: (0, n)),
        out_shape=jax.ShapeDtypeStruct((m_loc, n_tot), jnp.float32),
        scratch_shapes=[
            pltpu.VMEM((m_tot, k_loc), jnp.float8_e4m3fn),
            pltpu.VMEM((N_DEV, m_loc, k_loc), jnp.float8_e4m3fn),
            pltpu.SemaphoreType.DMA((N_DEV,)),
            pltpu.SemaphoreType.DMA((N_DEV,)),
        ],
        compiler_params=pltpu.CompilerParams(
            dimension_semantics=("arbitrary", "arbitrary"),
        ),
    )(x, w_mat, scale_x, scale_w)
